# baseline (speedup 1.0000x reference)
import functools
import math

import jax
import jax.numpy as jnp
from jax import lax
from jax.experimental import pallas as pl
from jax.experimental.pallas import tpu as pltpu
from jax.experimental.pallas import tpu_sc as plsc

_B, _N, _M = 16, 500, 64
_L = 16
_NCHUNK = _N // _L
_TAIL = _N - _NCHUNK * _L
_TBASE = _N - _L


def _sc_body(x_hbm, pb_hbm, tb_hbm, lab_hbm, src_hbm, tgt_hbm,
             part_hbm, out_hbm,
             xv, zv, pbv, tbv, labv, srcv, tgtv, accv, allv, outv, sem):
    cid = lax.axis_index("c")
    sid = lax.axis_index("s")
    row = sid
    zero = jnp.zeros((_L,), jnp.float32)
    lane = lax.iota(jnp.int32, _L)

    cp_src = pltpu.async_copy(src_hbm.at[row], srcv, sem)
    cp_tgt = pltpu.async_copy(tgt_hbm.at[row], tgtv, sem)

    @pl.when(cid == 0)
    def _():
        cp_x = pltpu.async_copy(x_hbm.at[row], xv, sem)
        cp_lab = pltpu.async_copy(lab_hbm.at[row], labv, sem)
        for k in range(_N // _L + 1):
            zv[pl.ds(k * _L, _L)] = zero
        cp_src.wait()
        cp_tgt.wait()
        cp_x.wait()
        cp_lab.wait()

        for k in range(_M // _L):
            svec = srcv[pl.ds(k * _L, _L)]
            tvec = tgtv[pl.ds(k * _L, _L)]
            lab = plsc.load_gather(labv, [tvec]).astype(jnp.float32)
            plsc.store_scatter(zv, [svec], lab)

        def bce(x, z):
            u = jnp.exp(-jnp.abs(x))
            s = u / (u + 2.0)
            s2 = s * s
            lg = s * (2.0 + s2 * (2.0 / 3.0
                                  + s2 * (2.0 / 5.0 + s2 * (2.0 / 7.0))))
            return jnp.maximum(x, 0.0) - x * z + lg

        acc = zero
        for k in range(_NCHUNK):
            acc = acc + bce(xv[0, pl.ds(k * _L, _L)], zv[pl.ds(k * _L, _L)])
        tail = bce(xv[0, pl.ds(_TBASE, _L)], zv[pl.ds(_TBASE, _L)])
        accv[...] = acc + jnp.where(lane >= _L - _TAIL, tail, 0.0)

    @pl.when(cid == 1)
    def _():
        cps = [pltpu.async_copy(pb_hbm.at[c, pl.ds(row, 1)], pbv.at[c], sem)
               for c in range(3)]
        cps += [pltpu.async_copy(tb_hbm.at[c, pl.ds(row, 1)], tbv.at[c], sem)
                for c in range(3)]
        cp_src.wait()
        cp_tgt.wait()
        for cp in cps:
            cp.wait()

        acc = zero
        for k in range(_M // _L):
            svec = srcv[pl.ds(k * _L, _L)]
            tvec = tgtv[pl.ds(k * _L, _L)]
            zvec = jnp.zeros((_L,), jnp.int32)
            for c in range(3):
                cvec = jnp.full((_L,), c, jnp.int32)
                sp = plsc.load_gather(pbv, [cvec, zvec, svec])
                tp = plsc.load_gather(tbv, [cvec, zvec, tvec])
                acc = acc + jnp.abs(sp - tp)
        accv[...] = acc

    pltpu.sync_copy(accv, part_hbm.at[cid, sid])
    plsc.subcore_barrier()

    @pl.when(sid == 0)
    def _():
        pltpu.sync_copy(part_hbm.at[cid], allv)
        tot = jnp.zeros((_L,), jnp.float32)
        for s_ in range(_B):
            tot = tot + allv[s_, :]
        scale = jnp.where(cid == 0, 1.0 / (_B * _N), 1.0 / (_B * _M))
        loss = jnp.sum(tot) * scale
        outv[...] = jnp.where(lane == 0, loss, 0.0)
        pltpu.sync_copy(outv, out_hbm.at[pl.ds(cid * _L, _L)])


_SCRATCH = [
    pltpu.VMEM((1, _N), jnp.float32),
    pltpu.VMEM((_N + _L, ), jnp.float32),
    pltpu.VMEM((3, 1, _N), jnp.float32),
    pltpu.VMEM((3, 1, _M), jnp.float32),
    pltpu.VMEM((_M,), jnp.int32),
    pltpu.VMEM((_M,), jnp.int32),
    pltpu.VMEM((_M,), jnp.int32),
    pltpu.VMEM((_L,), jnp.float32),
    pltpu.VMEM((_B, _L), jnp.float32),
    pltpu.VMEM((_L,), jnp.float32),
    pltpu.SemaphoreType.DMA,
]


def _make_kernel(interpret=False):
    return pl.kernel(
        _sc_body,
        out_type=(jax.ShapeDtypeStruct((2, _B, _L), jnp.float32),
                  jax.ShapeDtypeStruct((2 * _L,), jnp.float32)),
        mesh=plsc.VectorSubcoreMesh(
            core_axis_name="c", subcore_axis_name="s",
            num_cores=2, num_subcores=16),
        scratch_types=_SCRATCH,
        compiler_params=pltpu.CompilerParams(needs_layout_passes=False),
        interpret=interpret,
    )


def kernel(pred_logits, pred_boxes, tgt_boxes, tgt_labels, src_idx, tgt_idx):
    xT = jnp.transpose(pred_logits.astype(jnp.float32), (0, 2, 1))
    _, out = _make_kernel()(
        xT,
        jnp.transpose(pred_boxes.astype(jnp.float32), (2, 0, 1)),
        jnp.transpose(tgt_boxes.astype(jnp.float32), (2, 0, 1)),
        tgt_labels.astype(jnp.int32),
        src_idx.astype(jnp.int32),
        tgt_idx.astype(jnp.int32),
    )
    return (out[0], out[_L])


# speedup vs baseline: 1.0204x; 1.0204x over previous
import functools
import math

import jax
import jax.numpy as jnp
from jax import lax
from jax.experimental import pallas as pl
from jax.experimental.pallas import tpu as pltpu
from jax.experimental.pallas import tpu_sc as plsc

_B, _N, _M = 16, 500, 64
_L = 16
_NCHUNK = _N // _L
_TAIL = _N - _NCHUNK * _L
_TBASE = _N - _L


def _sc_body(x_hbm, pb_hbm, tb_hbm, lab_hbm, src_hbm, tgt_hbm,
             part_hbm, out_hbm,
             xv, zv, pbv, tbv, labv, srcv, tgtv, accv, allv, outv, sem):
    cid = lax.axis_index("c")
    sid = lax.axis_index("s")
    row = sid
    zero = jnp.zeros((_L,), jnp.float32)
    lane = lax.iota(jnp.int32, _L)

    cp_src = pltpu.async_copy(src_hbm.at[row], srcv, sem)
    cp_tgt = pltpu.async_copy(tgt_hbm.at[row], tgtv, sem)

    @pl.when(cid == 0)
    def _():
        cp_x = pltpu.async_copy(x_hbm.at[row], xv, sem)
        cp_lab = pltpu.async_copy(lab_hbm.at[row], labv, sem)
        def zfill(k, _):
            zv[pl.ds(k * _L, _L)] = zero
            return 0
        lax.fori_loop(0, _N // _L + 1, zfill, 0, unroll=False)
        cp_src.wait()
        cp_tgt.wait()
        cp_x.wait()
        cp_lab.wait()

        for k in range(_M // _L):
            svec = srcv[pl.ds(k * _L, _L)]
            tvec = tgtv[pl.ds(k * _L, _L)]
            lab = plsc.load_gather(labv, [tvec]).astype(jnp.float32)
            plsc.store_scatter(zv, [svec], lab)

        def bce(x, z):
            u = jnp.exp(-jnp.abs(x))
            s = u / (u + 2.0)
            s2 = s * s
            lg = s * (2.0 + s2 * (2.0 / 3.0
                                  + s2 * (2.0 / 5.0 + s2 * (2.0 / 7.0))))
            return jnp.maximum(x, 0.0) - x * z + lg

        def step(k, acc):
            return acc + bce(xv[0, pl.ds(k * _L, _L)], zv[pl.ds(k * _L, _L)])
        acc = lax.fori_loop(0, _NCHUNK, step, zero, unroll=False)
        tail = bce(xv[0, pl.ds(_TBASE, _L)], zv[pl.ds(_TBASE, _L)])
        accv[...] = acc + jnp.where(lane >= _L - _TAIL, tail, 0.0)

    @pl.when(cid == 1)
    def _():
        cps = [pltpu.async_copy(pb_hbm.at[c, pl.ds(row, 1)], pbv.at[c], sem)
               for c in range(3)]
        cps += [pltpu.async_copy(tb_hbm.at[c, pl.ds(row, 1)], tbv.at[c], sem)
                for c in range(3)]
        cp_src.wait()
        cp_tgt.wait()
        for cp in cps:
            cp.wait()

        acc = zero
        for k in range(_M // _L):
            svec = srcv[pl.ds(k * _L, _L)]
            tvec = tgtv[pl.ds(k * _L, _L)]
            zvec = jnp.zeros((_L,), jnp.int32)
            for c in range(3):
                cvec = jnp.full((_L,), c, jnp.int32)
                sp = plsc.load_gather(pbv, [cvec, zvec, svec])
                tp = plsc.load_gather(tbv, [cvec, zvec, tvec])
                acc = acc + jnp.abs(sp - tp)
        accv[...] = acc

    pltpu.sync_copy(accv, part_hbm.at[cid, sid])
    plsc.subcore_barrier()

    @pl.when(sid == 0)
    def _():
        pltpu.sync_copy(part_hbm.at[cid], allv)
        def red(s_, tot):
            return tot + allv[s_, :]
        tot = lax.fori_loop(0, _B, red, jnp.zeros((_L,), jnp.float32),
                            unroll=False)
        scale = jnp.where(cid == 0, 1.0 / (_B * _N), 1.0 / (_B * _M))
        loss = jnp.sum(tot) * scale
        outv[...] = jnp.where(lane == 0, loss, 0.0)
        pltpu.sync_copy(outv, out_hbm.at[pl.ds(cid * _L, _L)])


_SCRATCH = [
    pltpu.VMEM((1, _N), jnp.float32),
    pltpu.VMEM((_N + _L, ), jnp.float32),
    pltpu.VMEM((3, 1, _N), jnp.float32),
    pltpu.VMEM((3, 1, _M), jnp.float32),
    pltpu.VMEM((_M,), jnp.int32),
    pltpu.VMEM((_M,), jnp.int32),
    pltpu.VMEM((_M,), jnp.int32),
    pltpu.VMEM((_L,), jnp.float32),
    pltpu.VMEM((_B, _L), jnp.float32),
    pltpu.VMEM((_L,), jnp.float32),
    pltpu.SemaphoreType.DMA,
]


def _make_kernel(interpret=False):
    return pl.kernel(
        _sc_body,
        out_type=(jax.ShapeDtypeStruct((2, _B, _L), jnp.float32),
                  jax.ShapeDtypeStruct((2 * _L,), jnp.float32)),
        mesh=plsc.VectorSubcoreMesh(
            core_axis_name="c", subcore_axis_name="s",
            num_cores=2, num_subcores=16),
        scratch_types=_SCRATCH,
        compiler_params=pltpu.CompilerParams(needs_layout_passes=False),
        interpret=interpret,
    )


def kernel(pred_logits, pred_boxes, tgt_boxes, tgt_labels, src_idx, tgt_idx):
    xT = jnp.transpose(pred_logits.astype(jnp.float32), (0, 2, 1))
    _, out = _make_kernel()(
        xT,
        jnp.transpose(pred_boxes.astype(jnp.float32), (2, 0, 1)),
        jnp.transpose(tgt_boxes.astype(jnp.float32), (2, 0, 1)),
        tgt_labels.astype(jnp.int32),
        src_idx.astype(jnp.int32),
        tgt_idx.astype(jnp.int32),
    )
    return (out[0], out[_L])


# speedup vs baseline: 1.0958x; 1.0739x over previous
import jax
import jax.numpy as jnp
from jax import lax
from jax.experimental import pallas as pl
from jax.experimental.pallas import tpu as pltpu
from jax.experimental.pallas import tpu_sc as plsc

_B, _N, _M = 16, 500, 64
_L = 16
_NCHUNK = _N // _L
_TAIL = _N - _NCHUNK * _L
_TBASE = _N - _L


def _sc_body(x_hbm, pb_hbm, tb_hbm, lab_hbm, src_hbm, tgt_hbm,
             part_hbm, out_hbm,
             xv, zv, pbv, tbv, labv, srcv, tgtv, accv, allv, outv, sem):
    sid = lax.axis_index("s")
    row = sid
    zero = jnp.zeros((_L,), jnp.float32)
    lane = lax.iota(jnp.int32, _L)

    cps = [
        pltpu.async_copy(src_hbm.at[row], srcv, sem),
        pltpu.async_copy(tgt_hbm.at[row], tgtv, sem),
        pltpu.async_copy(x_hbm.at[row], xv, sem),
        pltpu.async_copy(lab_hbm.at[row], labv, sem),
    ]
    cps += [pltpu.async_copy(pb_hbm.at[c, pl.ds(row, 1)], pbv.at[c], sem)
            for c in range(3)]
    cps += [pltpu.async_copy(tb_hbm.at[c, pl.ds(row, 1)], tbv.at[c], sem)
            for c in range(3)]

    def zfill(k, _):
        zv[pl.ds(k * _L, _L)] = zero
        return 0
    lax.fori_loop(0, _N // _L + 1, zfill, 0, unroll=False)

    for cp in cps:
        cp.wait()

    for k in range(_M // _L):
        svec = srcv[pl.ds(k * _L, _L)]
        tvec = tgtv[pl.ds(k * _L, _L)]
        lab = plsc.load_gather(labv, [tvec]).astype(jnp.float32)
        plsc.store_scatter(zv, [svec], lab)

    def bce(x, z):
        u = jnp.exp(-jnp.abs(x))
        s = u / (u + 2.0)
        s2 = s * s
        lg = s * (2.0 + s2 * (2.0 / 3.0 + s2 * (2.0 / 5.0 + s2 * (2.0 / 7.0))))
        return jnp.maximum(x, 0.0) - x * z + lg

    def step(k, acc):
        return acc + bce(xv[0, pl.ds(k * _L, _L)], zv[pl.ds(k * _L, _L)])
    acc_ce = lax.fori_loop(0, _NCHUNK, step, zero, unroll=False)
    tail = bce(xv[0, pl.ds(_TBASE, _L)], zv[pl.ds(_TBASE, _L)])
    acc_ce = acc_ce + jnp.where(lane >= _L - _TAIL, tail, 0.0)

    acc_bb = zero
    zvec = jnp.zeros((_L,), jnp.int32)
    for k in range(_M // _L):
        svec = srcv[pl.ds(k * _L, _L)]
        tvec = tgtv[pl.ds(k * _L, _L)]
        for c in range(3):
            cvec = jnp.full((_L,), c, jnp.int32)
            sp = plsc.load_gather(pbv, [cvec, zvec, svec])
            tp = plsc.load_gather(tbv, [cvec, zvec, tvec])
            acc_bb = acc_bb + jnp.abs(sp - tp)

    accv[0, :] = acc_ce
    accv[1, :] = acc_bb
    pltpu.sync_copy(accv, part_hbm.at[sid])
    plsc.subcore_barrier()

    @pl.when(sid == 0)
    def _():
        pltpu.sync_copy(part_hbm, allv)

        def red(s_, carry):
            ce, bb = carry
            return (ce + allv[s_, 0, :], bb + allv[s_, 1, :])
        ce, bb = lax.fori_loop(0, _B, red, (zero, zero), unroll=False)
        loss_ce = jnp.sum(ce) * (1.0 / (_B * _N))
        loss_bb = jnp.sum(bb) * (1.0 / (_B * _M))
        outv[...] = jnp.where(lane == 0, loss_ce,
                              jnp.where(lane == 1, loss_bb, 0.0))
        pltpu.sync_copy(outv, out_hbm)


_SCRATCH = [
    pltpu.VMEM((1, _N), jnp.float32),
    pltpu.VMEM((_N + _L,), jnp.float32),
    pltpu.VMEM((3, 1, _N), jnp.float32),
    pltpu.VMEM((3, 1, _M), jnp.float32),
    pltpu.VMEM((_M,), jnp.int32),
    pltpu.VMEM((_M,), jnp.int32),
    pltpu.VMEM((_M,), jnp.int32),
    pltpu.VMEM((2, _L), jnp.float32),
    pltpu.VMEM((_B, 2, _L), jnp.float32),
    pltpu.VMEM((_L,), jnp.float32),
    pltpu.SemaphoreType.DMA,
]


def _make_kernel():
    return pl.kernel(
        _sc_body,
        out_type=(jax.ShapeDtypeStruct((_B, 2, _L), jnp.float32),
                  jax.ShapeDtypeStruct((_L,), jnp.float32)),
        mesh=plsc.VectorSubcoreMesh(
            core_axis_name="c", subcore_axis_name="s",
            num_cores=1, num_subcores=16),
        scratch_types=_SCRATCH,
        compiler_params=pltpu.CompilerParams(needs_layout_passes=False),
    )


def kernel(pred_logits, pred_boxes, tgt_boxes, tgt_labels, src_idx, tgt_idx):
    xT = jnp.transpose(pred_logits.astype(jnp.float32), (0, 2, 1))
    _, out = _make_kernel()(
        xT,
        jnp.transpose(pred_boxes.astype(jnp.float32), (2, 0, 1)),
        jnp.transpose(tgt_boxes.astype(jnp.float32), (2, 0, 1)),
        tgt_labels.astype(jnp.int32),
        src_idx.astype(jnp.int32),
        tgt_idx.astype(jnp.int32),
    )
    return (out[0], out[1])
